# Initial kernel scaffold; baseline (speedup 1.0000x reference)
import jax
import jax.numpy as jnp
from jax.experimental import pallas as pl


def kernel(x, edge_index, W1l, b1, W1r, W2l, b2, W2r):
    raise NotImplementedError("write your pallas kernel here")


# speedup vs baseline: 8.7421x; 8.7421x over previous
import functools

import jax
import jax.numpy as jnp
from jax import lax
from jax.experimental import pallas as pl
from jax.experimental.pallas import tpu as pltpu
from jax.experimental.pallas import tpu_sc as plsc

NC = 2
NS = 16
NW = NC * NS
CH = 128
RB = 1024


def _make_sc_agg(n_pad, d, n_chunks, with_counts):
  stripe = n_pad // NS
  mesh = plsc.VectorSubcoreMesh(core_axis_name="c", subcore_axis_name="s")

  out_type = [jax.ShapeDtypeStruct((NC, n_pad, d), jnp.float32)]
  scratch = [
      pltpu.VMEM((n_chunks, CH), jnp.int32),
      pltpu.VMEM((n_chunks, CH), jnp.int32),
      pltpu.VMEM((CH, d), jnp.float32),
      pltpu.VMEM_SHARED((n_pad, d), jnp.float32),
      pltpu.SemaphoreType.DMA,
  ]
  if with_counts:
    out_type.append(jax.ShapeDtypeStruct((NC, n_pad), jnp.float32))
    scratch += [
        pltpu.VMEM((CH,), jnp.float32),
        pltpu.VMEM_SHARED((n_pad,), jnp.float32),
    ]

  @functools.partial(pl.kernel, mesh=mesh, out_type=tuple(out_type),
                     scratch_types=scratch)
  def agg(*refs):
    if with_counts:
      (x_hbm, src_hbm, dst_hbm, z2_hbm, z1_hbm, one_hbm,
       s_hbm, c_hbm, src_v, dst_v, rows_v, acc_sh, sem,
       ones_v, cnt_sh) = refs
    else:
      (x_hbm, src_hbm, dst_hbm, z2_hbm,
       s_hbm, src_v, dst_v, rows_v, acc_sh, sem) = refs

    cid = lax.axis_index("c")
    sid = lax.axis_index("s")
    wid = sid * NC + cid
    row0 = sid * stripe

    pltpu.sync_copy(src_hbm.at[wid], src_v)
    pltpu.sync_copy(dst_hbm.at[wid], dst_v)
    pltpu.sync_copy(z2_hbm.at[pl.ds(row0, stripe)],
                    acc_sh.at[pl.ds(row0, stripe)])
    if with_counts:
      pltpu.sync_copy(one_hbm, ones_v)
      pltpu.sync_copy(z1_hbm.at[pl.ds(row0, stripe)],
                      cnt_sh.at[pl.ds(row0, stripe)])
    plsc.subcore_barrier()

    def step(j, carry):
      idx = src_v.at[j]
      pltpu.async_copy(x_hbm.at[idx], rows_v, sem).wait()
      pltpu.sync_copy(rows_v, acc_sh.at[dst_v.at[j]], add=True)
      if with_counts:
        pltpu.sync_copy(ones_v, cnt_sh.at[dst_v.at[j]], add=True)
      return carry

    lax.fori_loop(0, n_chunks, step, 0)
    plsc.subcore_barrier()

    pltpu.sync_copy(acc_sh.at[pl.ds(row0, stripe)],
                    s_hbm.at[cid, pl.ds(row0, stripe)])
    if with_counts:
      pltpu.sync_copy(cnt_sh.at[pl.ds(row0, stripe)],
                      c_hbm.at[cid, pl.ds(row0, stripe)])

  return agg


def _tc1_body(s_ref, c_ref, x_ref, w1l_ref, b1_ref, w1r_ref,
              w2r_ref, b2_ref, h_ref, r_ref):
  c = c_ref[...]
  tot = jnp.maximum(c[:, 0:1] + c[:, 1:2], 1.0)
  mean = (s_ref[0] + s_ref[1]) * (1.0 / tot)
  xw = jnp.dot(x_ref[...], w1r_ref[...], preferred_element_type=jnp.float32)
  mw = jnp.dot(mean, w1l_ref[...], preferred_element_type=jnp.float32)
  h = jnp.maximum(mw + xw + b1_ref[...], 0.0)
  h_ref[...] = h
  r_ref[...] = (
      jnp.dot(h, w2r_ref[...], preferred_element_type=jnp.float32)
      + b2_ref[...])


def _tc2_body(s_ref, c_ref, w2l_ref, r_ref, z_ref):
  c = c_ref[...]
  tot = jnp.maximum(c[:, 0:1] + c[:, 1:2], 1.0)
  mean2 = (s_ref[0] + s_ref[1]) * (1.0 / tot)
  z_ref[...] = (
      jnp.dot(mean2, w2l_ref[...], preferred_element_type=jnp.float32)
      + r_ref[...])


def kernel(x, edge_index, W1l, b1, W1r, W2l, b2, W2r):
  n, d_in = x.shape
  e = edge_index.shape[1]
  d_h = W1l.shape[0]
  d_out = W2l.shape[0]

  n_pad = ((n + RB - 1) // RB) * RB
  n_chunks = -(-e // (NW * CH))
  e_pad = NW * n_chunks * CH

  src = edge_index[0].astype(jnp.int32)
  dst = edge_index[1].astype(jnp.int32)
  pad = e_pad - e
  pad_i = jnp.arange(pad, dtype=jnp.int32)
  src_p = jnp.concatenate([src, pad_i % n])
  dst_p = jnp.concatenate([dst, n + pad_i % (n_pad - n)])
  src3 = src_p.reshape(NW, n_chunks, CH)
  dst3 = dst_p.reshape(NW, n_chunks, CH)

  x_pad = jnp.pad(x.astype(jnp.float32), ((0, n_pad - n), (0, 0)))
  z2 = jnp.zeros((n_pad, d_in), jnp.float32)
  z1 = jnp.zeros((n_pad,), jnp.float32)
  ones = jnp.ones((CH,), jnp.float32)

  agg1 = _make_sc_agg(n_pad, d_in, n_chunks, with_counts=True)
  s1p, cntp = agg1(x_pad, src3, dst3, z2, z1, ones)
  cnt2 = cntp.T

  grid = (n_pad // RB,)
  w1l_t = W1l.T.astype(jnp.float32)
  w1r_t = W1r.T.astype(jnp.float32)
  w2l_t = W2l.T.astype(jnp.float32)
  w2r_t = W2r.T.astype(jnp.float32)
  b1_2 = b1.astype(jnp.float32).reshape(1, d_h)
  b2_2 = b2.astype(jnp.float32).reshape(1, d_out)

  h, r = pl.pallas_call(
      _tc1_body,
      grid=grid,
      in_specs=[
          pl.BlockSpec((NC, RB, d_in), lambda i: (0, i, 0)),
          pl.BlockSpec((RB, NC), lambda i: (i, 0)),
          pl.BlockSpec((RB, d_in), lambda i: (i, 0)),
          pl.BlockSpec((d_in, d_h), lambda i: (0, 0)),
          pl.BlockSpec((1, d_h), lambda i: (0, 0)),
          pl.BlockSpec((d_in, d_h), lambda i: (0, 0)),
          pl.BlockSpec((d_h, d_out), lambda i: (0, 0)),
          pl.BlockSpec((1, d_out), lambda i: (0, 0)),
      ],
      out_specs=[
          pl.BlockSpec((RB, d_h), lambda i: (i, 0)),
          pl.BlockSpec((RB, d_out), lambda i: (i, 0)),
      ],
      out_shape=[
          jax.ShapeDtypeStruct((n_pad, d_h), jnp.float32),
          jax.ShapeDtypeStruct((n_pad, d_out), jnp.float32),
      ],
  )(s1p, cnt2, x_pad, w1l_t, b1_2, w1r_t, w2r_t, b2_2)

  agg2 = _make_sc_agg(n_pad, d_h, n_chunks, with_counts=False)
  (s2p,) = agg2(h, src3, dst3, z2)

  z = pl.pallas_call(
      _tc2_body,
      grid=grid,
      in_specs=[
          pl.BlockSpec((NC, RB, d_h), lambda i: (0, i, 0)),
          pl.BlockSpec((RB, NC), lambda i: (i, 0)),
          pl.BlockSpec((d_h, d_out), lambda i: (0, 0)),
          pl.BlockSpec((RB, d_out), lambda i: (i, 0)),
      ],
      out_specs=pl.BlockSpec((RB, d_out), lambda i: (i, 0)),
      out_shape=jax.ShapeDtypeStruct((n_pad, d_out), jnp.float32),
  )(s2p, cnt2, w2l_t, r)

  return z[:n]
